# baseline (speedup 1.0000x reference)
import jax
import jax.numpy as jnp
from jax.experimental import pallas as pl
from jax.experimental.pallas import tpu as pltpu


def _xattn_kernel(in_ref, xt_ref, et_ref, wv_ref, wo_ref, ws_ref, out_ref,
                  sem1, sem2):
    nb = pl.num_programs(1)
    j = pl.program_id(1)
    cp1 = pltpu.make_async_copy(in_ref.at[0, 1], out_ref.at[0, 1], sem1)
    cp2 = pltpu.make_async_copy(in_ref.at[0, 2], out_ref.at[0, 2], sem2)
    cp1.start()
    cp2.start()
    x = in_ref[0, 0]
    e = in_ref[0, 2]
    wv = wv_ref[...]

    v = (jnp.dot(x, wv[:3], preferred_element_type=jnp.float32)
         + jnp.dot(e, wv[3:], preferred_element_type=jnp.float32))

    jp = jnp.where(j == 0, nb - 1, j - 1)
    xt = xt_ref[0, pl.ds(jp, 1), :]
    et = et_ref[0, pl.ds(jp, 1), :]
    vprev = (jnp.dot(xt, wv[:3], preferred_element_type=jnp.float32)
             + jnp.dot(et, wv[3:], preferred_element_type=jnp.float32))

    rolled = pltpu.roll(v, 1, 0)
    row = jax.lax.broadcasted_iota(jnp.int32, rolled.shape, 0)
    vb = jnp.where(row == 0, jnp.broadcast_to(vprev, rolled.shape), rolled)

    out0 = (jnp.dot(x, ws_ref[...], preferred_element_type=jnp.float32)
            + jnp.dot(vb, wo_ref[...], preferred_element_type=jnp.float32))
    out_ref[0, 0] = out0
    cp1.wait()
    cp2.wait()


def kernel(input_data, Wq, Wk, Wv, Wo, Wself):
    B, C, N, D = input_data.shape
    R = 5000
    nb = N // R

    xt = input_data[:, 0, R - 1::R, :]
    et = input_data[:, 2, R - 1::R, :]

    return pl.pallas_call(
        _xattn_kernel,
        grid=(B, nb),
        in_specs=[
            pl.BlockSpec((1, C, R, D), lambda b, j: (b, 0, j, 0)),
            pl.BlockSpec((1, nb, D), lambda b, j: (b, 0, 0)),
            pl.BlockSpec((1, nb, D), lambda b, j: (b, 0, 0)),
            pl.BlockSpec((2 * D, 1), lambda b, j: (0, 0)),
            pl.BlockSpec((1, D), lambda b, j: (0, 0)),
            pl.BlockSpec((D, D), lambda b, j: (0, 0)),
        ],
        out_specs=pl.BlockSpec((1, C, R, D), lambda b, j: (b, 0, j, 0)),
        out_shape=jax.ShapeDtypeStruct((B, C, N, D), jnp.float32),
        scratch_shapes=[pltpu.SemaphoreType.DMA, pltpu.SemaphoreType.DMA],
    )(input_data, xt, et, Wv, Wo, Wself)


# speedup vs baseline: 1.0677x; 1.0677x over previous
import jax
import jax.numpy as jnp
from jax.experimental import pallas as pl
from jax.experimental.pallas import tpu as pltpu


def _xattn_kernel(x_ref, m_ref, e_ref, xt_ref, et_ref, wv_ref, wo_ref,
                  ws_ref, out_ref):
    nb = pl.num_programs(1)
    j = pl.program_id(1)
    x = x_ref[0, 0]
    e = e_ref[0, 0]
    wv = wv_ref[...]

    v = (jnp.dot(x, wv[:3], preferred_element_type=jnp.float32)
         + jnp.dot(e, wv[3:], preferred_element_type=jnp.float32))

    jp = jnp.where(j == 0, nb - 1, j - 1)
    xt = xt_ref[0, pl.ds(jp, 1), :]
    et = et_ref[0, pl.ds(jp, 1), :]
    vprev = (jnp.dot(xt, wv[:3], preferred_element_type=jnp.float32)
             + jnp.dot(et, wv[3:], preferred_element_type=jnp.float32))

    rolled = pltpu.roll(v, 1, 0)
    row = jax.lax.broadcasted_iota(jnp.int32, rolled.shape, 0)
    vb = jnp.where(row == 0, jnp.broadcast_to(vprev, rolled.shape), rolled)

    out0 = (jnp.dot(x, ws_ref[...], preferred_element_type=jnp.float32)
            + jnp.dot(vb, wo_ref[...], preferred_element_type=jnp.float32))
    out_ref[0, 0] = out0
    out_ref[0, 1] = m_ref[0, 0]
    out_ref[0, 2] = e


def kernel(input_data, Wq, Wk, Wv, Wo, Wself):
    B, C, N, D = input_data.shape
    R = 2000
    nb = N // R

    xt = input_data[:, 0, R - 1::R, :]
    et = input_data[:, 2, R - 1::R, :]

    return pl.pallas_call(
        _xattn_kernel,
        grid=(B, nb),
        in_specs=[
            pl.BlockSpec((1, 1, R, D), lambda b, j: (b, 0, j, 0)),
            pl.BlockSpec((1, 1, R, D), lambda b, j: (b, 1, j, 0)),
            pl.BlockSpec((1, 1, R, D), lambda b, j: (b, 2, j, 0)),
            pl.BlockSpec((1, nb, D), lambda b, j: (b, 0, 0)),
            pl.BlockSpec((1, nb, D), lambda b, j: (b, 0, 0)),
            pl.BlockSpec((2 * D, 1), lambda b, j: (0, 0)),
            pl.BlockSpec((1, D), lambda b, j: (0, 0)),
            pl.BlockSpec((D, D), lambda b, j: (0, 0)),
        ],
        out_specs=pl.BlockSpec((1, C, R, D), lambda b, j: (b, 0, j, 0)),
        out_shape=jax.ShapeDtypeStruct((B, C, N, D), jnp.float32),
    )(input_data, input_data, input_data, xt, et, Wv, Wo, Wself)


# speedup vs baseline: 1.1237x; 1.0524x over previous
import jax
import jax.numpy as jnp
from jax.experimental import pallas as pl
from jax.experimental.pallas import tpu as pltpu


def _xattn_kernel(x_ref, m_ref, e_ref, tail_ref, wxo_ref, weo_ref, ws_ref,
                  out_ref, carry_ref):
    j = pl.program_id(1)
    x = x_ref[0, 0]
    e = e_ref[0, 0]
    wxo = wxo_ref[...]
    weo = weo_ref[...]

    vexp = (jnp.dot(x, wxo, preferred_element_type=jnp.float32)
            + jnp.dot(e, weo, preferred_element_type=jnp.float32))

    tx = tail_ref[0, 0:1, :]
    te = tail_ref[0, 2:3, :]
    tail_vexp = (jnp.dot(tx, wxo, preferred_element_type=jnp.float32)
                 + jnp.dot(te, weo, preferred_element_type=jnp.float32))
    vprev = jnp.where(j == 0, tail_vexp, carry_ref[0:1, 0:3])
    carry_ref[0:1, 0:3] = vexp[-1:, :]

    rolled = pltpu.roll(vexp, 1, 0)
    row = jax.lax.broadcasted_iota(jnp.int32, rolled.shape, 0)
    w = jnp.where(row == 0, jnp.broadcast_to(vprev, rolled.shape), rolled)

    out_ref[0, 0] = w + jnp.dot(x, ws_ref[...],
                                preferred_element_type=jnp.float32)
    out_ref[0, 1] = m_ref[0, 0]
    out_ref[0, 2] = e


def kernel(input_data, Wq, Wk, Wv, Wo, Wself):
    B, C, N, D = input_data.shape
    R = 2000
    nb = N // R

    tail = input_data[:, :, N - 1, :]
    wxo = Wv[:D] @ Wo
    weo = Wv[D:] @ Wo

    return pl.pallas_call(
        _xattn_kernel,
        grid=(B, nb),
        in_specs=[
            pl.BlockSpec((1, 1, R, D), lambda b, j: (b, 0, j, 0)),
            pl.BlockSpec((1, 1, R, D), lambda b, j: (b, 1, j, 0)),
            pl.BlockSpec((1, 1, R, D), lambda b, j: (b, 2, j, 0)),
            pl.BlockSpec((1, C, D), lambda b, j: (b, 0, 0)),
            pl.BlockSpec((D, D), lambda b, j: (0, 0)),
            pl.BlockSpec((D, D), lambda b, j: (0, 0)),
            pl.BlockSpec((D, D), lambda b, j: (0, 0)),
        ],
        out_specs=pl.BlockSpec((1, C, R, D), lambda b, j: (b, 0, j, 0)),
        out_shape=jax.ShapeDtypeStruct((B, C, N, D), jnp.float32),
        scratch_shapes=[pltpu.VMEM((8, 128), jnp.float32)],
    )(input_data, input_data, input_data, tail, wxo, weo, Wself)
